# baseline (speedup 1.0000x reference)
import jax
import jax.numpy as jnp
import numpy as np
from jax.experimental import pallas as pl

_P = 4 * 512 * 512
_GROUP = 32
_ROWS = _P // _GROUP
_BLK = 1024
_GRID = _ROWS // _BLK


def _selection_matrices():
    s = np.zeros((384, 128), np.float32)
    t = np.zeros((128, 128), np.float32)
    for p in range(32):
        for c in range(3):
            s[12 * p + c, 4 * p + c] = 1.0
        for c in range(4):
            t[4 * p, 4 * p + c] = 1.0
    return jnp.asarray(s), jnp.asarray(t)


def _shader_body(colors_ref, pix_ref, s_ref, t_ref, out_ref):
    cb = colors_ref[...]
    rgb = jax.lax.dot_general(
        cb, s_ref[...], (((1,), (0,)), ((), ())),
        precision=jax.lax.Precision.HIGHEST,
        preferred_element_type=jnp.float32)
    bgf = (pix_ref[...] < 0).astype(jnp.float32)
    bgb = jax.lax.dot_general(
        bgf, t_ref[...], (((1,), (0,)), ((), ())),
        precision=jax.lax.Precision.HIGHEST,
        preferred_element_type=jnp.float32)
    lane = jax.lax.broadcasted_iota(jnp.int32, (1, 128), 1)
    alpha_lane = ((lane % 4) == 3).astype(jnp.float32)
    out_ref[...] = jnp.where(bgb > 0.5, 1.0 - alpha_lane, rgb + alpha_lane)


def kernel(colors, pix_to_face):
    n, h, w = colors.shape[0], colors.shape[1], colors.shape[2]
    colors2 = colors.reshape(_ROWS, _GROUP * 12)
    pix2 = pix_to_face.reshape(_ROWS, _GROUP * 4)
    s, t = _selection_matrices()
    out = pl.pallas_call(
        _shader_body,
        grid=(_GRID,),
        in_specs=[
            pl.BlockSpec((_BLK, 384), lambda i: (i, 0)),
            pl.BlockSpec((_BLK, 128), lambda i: (i, 0)),
            pl.BlockSpec((384, 128), lambda i: (0, 0)),
            pl.BlockSpec((128, 128), lambda i: (0, 0)),
        ],
        out_specs=pl.BlockSpec((_BLK, 128), lambda i: (i, 0)),
        out_shape=jax.ShapeDtypeStruct((_ROWS, 128), jnp.float32),
    )(colors2, pix2, s, t)
    return out.reshape(n, h, w, 4)


# speedup vs baseline: 491.9851x; 491.9851x over previous
import jax
import jax.numpy as jnp
from jax.experimental import pallas as pl

_NH = 4 * 512
_PW = 2
_HB = 128
_GRID = _NH // _HB


def _to_tiled_colors(colors):
    n, h, w, k, c = colors.shape
    t = colors.transpose(0, 1, 4, 3, 2)
    t = t.reshape(n, h, c, k, w // 128, 128)
    t = t.transpose(0, 1, 2, 4, 3, 5)
    return t.reshape(n * h, c, _PW, 8, 128)


def _to_tiled_pix(pix):
    n, h, w, k = pix.shape
    t = pix.transpose(0, 1, 3, 2)
    t = t.reshape(n, h, k, w // 128, 128)
    t = t.transpose(0, 1, 3, 2, 4)
    return t.reshape(n * h, _PW, 8, 128)


def _from_tiled_out(out, n, h, w):
    t = out.reshape(n, h, _PW, 2, 4, 128)
    t = t.transpose(0, 1, 2, 3, 5, 4)
    return t.reshape(n, h, w, 4)


def _shader_body(colors_ref, pix_ref, out_ref):
    cb = colors_ref[...]
    pb = pix_ref[...]
    c0 = cb.reshape(_HB, 3, _PW, 2, 4, 128)[:, :, :, :, 0, :]
    p0 = pb.reshape(_HB, _PW, 2, 4, 128)[:, :, :, 0, :]
    bg = p0 < 0
    one = jnp.float32(1.0)
    r = jnp.where(bg, one, c0[:, 0])
    g = jnp.where(bg, one, c0[:, 1])
    b = jnp.where(bg, one, c0[:, 2])
    a = jnp.where(bg, jnp.float32(0.0), one)
    out = jnp.stack([r, g, b, a], axis=-2)
    out_ref[...] = out.reshape(_HB, _PW, 8, 128)


def kernel(colors, pix_to_face):
    n, h, w = colors.shape[0], colors.shape[1], colors.shape[2]
    colors_t = _to_tiled_colors(colors)
    pix_t = _to_tiled_pix(pix_to_face)
    out = pl.pallas_call(
        _shader_body,
        grid=(_GRID,),
        in_specs=[
            pl.BlockSpec((_HB, 3, _PW, 8, 128), lambda i: (i, 0, 0, 0, 0)),
            pl.BlockSpec((_HB, _PW, 8, 128), lambda i: (i, 0, 0, 0)),
        ],
        out_specs=pl.BlockSpec((_HB, _PW, 8, 128), lambda i: (i, 0, 0, 0)),
        out_shape=jax.ShapeDtypeStruct((_NH, _PW, 8, 128), jnp.float32),
    )(colors_t, pix_t)
    return _from_tiled_out(out, n, h, w)
